# baseline (speedup 1.0000x reference)
import functools

import jax
import jax.numpy as jnp
from jax import lax
from jax.experimental import pallas as pl
from jax.experimental.pallas import tpu as pltpu
from jax.experimental.pallas import tpu_sc as plsc

VOCAB = 100000
FRONT = 256
EMBED = 64
HALF = VOCAB // 2

_NC = 2
_NS = 16
_NW = _NC * _NS


def _table_w1(table, W1, b1):
    BM = 2000
    grid = (HALF // BM,)

    def body(xlo_ref, xhi_ref, w1_ref, b1_ref, o_ref):
        def f(x):
            return lax.dot_general(x, w1_ref[:], (((1,), (1,)), ((), ())),
                                   preferred_element_type=jnp.float32) + b1_ref[:]
        o_ref[:, 0:EMBED] = f(xlo_ref[:])
        o_ref[:, EMBED:2 * EMBED] = f(xhi_ref[:])

    return pl.pallas_call(
        body,
        grid=grid,
        in_specs=[
            pl.BlockSpec((BM, FRONT), lambda i: (i, 0)),
            pl.BlockSpec((BM, FRONT), lambda i: (i + HALF // BM, 0)),
            pl.BlockSpec((EMBED, FRONT), lambda i: (0, 0)),
            pl.BlockSpec((1, EMBED), lambda i: (0, 0)),
        ],
        out_specs=pl.BlockSpec((BM, 2 * EMBED), lambda i: (i, 0)),
        out_shape=jax.ShapeDtypeStruct((HALF, 2 * EMBED), jnp.float32),
    )(table, table, W1, b1.reshape(1, EMBED))


def _sc_gather_wide(t1, idx_lo, idx_hi):
    lines = idx_lo.shape[0]
    l_per_w = lines // _NW
    C = min(800, l_per_w)
    n_sub = l_per_w // C
    n_ch = 2 * n_sub

    mesh = plsc.VectorSubcoreMesh(core_axis_name="c", subcore_axis_name="s")

    @functools.partial(
        pl.kernel,
        mesh=mesh,
        out_type=jax.ShapeDtypeStruct((lines, 2 * EMBED), jnp.float32),
        scratch_types=[
            pltpu.VMEM((C,), jnp.int32),
            pltpu.VMEM((C,), jnp.int32),
            pltpu.VMEM((C, EMBED), jnp.float32),
            pltpu.VMEM((C, EMBED), jnp.float32),
            pltpu.SemaphoreType.DMA,
            pltpu.SemaphoreType.DMA,
        ],
        compiler_params=pltpu.CompilerParams(use_tc_tiling_on_sc=False),
    )
    def k(t1_hbm, ilo_hbm, ihi_hbm, g_hbm, idx0, idx1, rows0, rows1,
          gsem, osem):
        wid = lax.axis_index("s") * _NC + lax.axis_index("c")
        lbase = wid * l_per_w
        idx_v = (idx0, idx1)
        rows_v = (rows0, rows1)
        srcs = ((ilo_hbm, 0),) * n_sub + ((ihi_hbm, 1),) * n_sub

        def chunk(i):
            src, half = srcs[i]
            line0 = lbase + (i % n_sub) * C
            return (src.at[pl.ds(line0, C)],
                    g_hbm.at[pl.ds(line0, C), pl.ds(half * EMBED, EMBED)])

        pltpu.sync_copy(chunk(0)[0], idx_v[0])
        gathers = [pltpu.async_copy(t1_hbm.at[idx_v[0]], rows_v[0], gsem)]
        outs = [None] * n_ch
        for i in range(n_ch):
            b = i % 2
            if i + 1 < n_ch:
                pltpu.sync_copy(chunk(i + 1)[0], idx_v[1 - b])
            gathers[i].wait()
            if i + 1 < n_ch:
                if i >= 1:
                    outs[i - 1].wait()
                gathers.append(pltpu.async_copy(
                    t1_hbm.at[idx_v[1 - b]], rows_v[1 - b], gsem))
            outs[i] = pltpu.async_copy(rows_v[b], chunk(i)[1], osem)
        outs[n_ch - 2].wait()
        outs[n_ch - 1].wait()

    return k(t1, idx_lo, idx_hi)


def _final_tc_part(g_part, W2, b2, B, L, Lp, s, prev):
    BH = B // 2

    def body(x_ref, w2_ref, b2_ref, *rest):
        o_ref = rest[-1]
        x = x_ref[:]
        g = x * 0.5 * (1.0 + lax.erf(x * (2.0 ** -0.5)))
        def f(gh):
            return lax.dot_general(w2_ref[:], gh, (((1,), (1,)), ((), ())),
                                   preferred_element_type=jnp.float32) + b2_ref[:]
        o_ref[0, :, 0:BH] = f(g[:, 0:EMBED])
        o_ref[0, :, BH:B] = f(g[:, EMBED:2 * EMBED])

    in_specs = [
        pl.BlockSpec((BH, 2 * EMBED), lambda l: (l, 0)),
        pl.BlockSpec((EMBED, EMBED), lambda l: (0, 0)),
        pl.BlockSpec((EMBED, 1), lambda l: (0, 0)),
    ]
    args = [g_part, W2, b2.reshape(EMBED, 1)]
    io_alias = {}
    if prev is not None:
        in_specs.append(pl.BlockSpec(memory_space=pl.ANY))
        args.append(prev)
        io_alias = {3: 0}
    return pl.pallas_call(
        body,
        grid=(Lp,),
        in_specs=in_specs,
        out_specs=pl.BlockSpec((1, EMBED, B), lambda l, s=s: (l + s * Lp, 0, 0)),
        out_shape=jax.ShapeDtypeStruct((L, EMBED, B), jnp.float32),
        input_output_aliases=io_alias,
    )(*args)


def kernel(idxs, table, W1, b1, W2, b2):
    B, L = idxs.shape
    t1w = _table_w1(table, W1, b1)
    t1 = t1w.reshape(VOCAB, EMBED)
    v = idxs.astype(jnp.int32)
    r = 2 * jnp.where(v < HALF, v, v - HALF) + (v >= HALF).astype(jnp.int32)
    rT = r.T
    idx_lo = rT[:, 0:B // 2].reshape(-1)
    idx_hi = rT[:, B // 2:B].reshape(-1)
    NSPLIT = 1
    Lp = L // NSPLIT
    lines_p = Lp * (B // 2)
    out_phys = None
    for s in range(NSPLIT):
        sl = slice(s * lines_p, (s + 1) * lines_p)
        g_s = _sc_gather_wide(t1, idx_lo[sl], idx_hi[sl])
        out_phys = _final_tc_part(g_s, W2, b2, B, L, Lp, s, out_phys)
    return jnp.transpose(out_phys, (2, 0, 1))


# speedup vs baseline: 1.1780x; 1.1780x over previous
import functools

import jax
import jax.numpy as jnp
from jax import lax
from jax.experimental import pallas as pl
from jax.experimental.pallas import tpu as pltpu
from jax.experimental.pallas import tpu_sc as plsc

VOCAB = 100000
FRONT = 256
EMBED = 64
HALF = VOCAB // 2

_NC = 2
_NS = 16
_NW = _NC * _NS


def _table_w1(table, W1, b1):
    BM = 2000
    grid = (HALF // BM,)

    def body(xlo_ref, xhi_ref, w1_ref, b1_ref, o_ref):
        def f(x):
            return lax.dot_general(x, w1_ref[:], (((1,), (1,)), ((), ())),
                                   preferred_element_type=jnp.float32) + b1_ref[:]
        o_ref[:, 0:EMBED] = f(xlo_ref[:])
        o_ref[:, EMBED:2 * EMBED] = f(xhi_ref[:])

    return pl.pallas_call(
        body,
        grid=grid,
        in_specs=[
            pl.BlockSpec((BM, FRONT), lambda i: (i, 0)),
            pl.BlockSpec((BM, FRONT), lambda i: (i + HALF // BM, 0)),
            pl.BlockSpec((EMBED, FRONT), lambda i: (0, 0)),
            pl.BlockSpec((1, EMBED), lambda i: (0, 0)),
        ],
        out_specs=pl.BlockSpec((BM, 2 * EMBED), lambda i: (i, 0)),
        out_shape=jax.ShapeDtypeStruct((HALF, 2 * EMBED), jnp.float32),
    )(table, table, W1, b1.reshape(1, EMBED))


def _sc_gather_wide(t1, idx_lo, idx_hi):
    lines = idx_lo.shape[0]
    l_per_w = lines // _NW
    C = min(1600, l_per_w)
    n_sub = l_per_w // C

    mesh = plsc.VectorSubcoreMesh(core_axis_name="c", subcore_axis_name="s")

    @functools.partial(
        pl.kernel,
        mesh=mesh,
        out_type=jax.ShapeDtypeStruct((lines, 2 * EMBED), jnp.float32),
        scratch_types=[
            pltpu.VMEM((C,), jnp.int32),
            pltpu.VMEM((C, EMBED), jnp.float32),
            pltpu.SemaphoreType.DMA,
        ],
        compiler_params=pltpu.CompilerParams(use_tc_tiling_on_sc=False),
    )
    def k(t1_hbm, ilo_hbm, ihi_hbm, g_hbm, idx_v, rows_v, sem):
        wid = lax.axis_index("s") * _NC + lax.axis_index("c")
        lbase = wid * l_per_w
        for half, src in ((0, ilo_hbm), (1, ihi_hbm)):
            for c in range(n_sub):
                line0 = lbase + c * C
                pltpu.sync_copy(src.at[pl.ds(line0, C)], idx_v)
                pltpu.async_copy(t1_hbm.at[idx_v], rows_v, sem).wait()
                pltpu.sync_copy(
                    rows_v,
                    g_hbm.at[pl.ds(line0, C), pl.ds(half * EMBED, EMBED)])

    return k(t1, idx_lo, idx_hi)


def _final_tc_part(g_part, W2, b2, B, L, Lp, s, prev):
    BH = B // 2
    P = 5

    def body(x_ref, w2_ref, b2_ref, *rest):
        o_ref = rest[-1]
        x = x_ref[:]
        g = x * 0.5 * (1.0 + lax.erf(x * (2.0 ** -0.5)))
        def f(gh):
            return lax.dot_general(w2_ref[:], gh, (((1,), (1,)), ((), ())),
                                   preferred_element_type=jnp.float32) + b2_ref[:]
        zlo = f(g[:, 0:EMBED])
        zhi = f(g[:, EMBED:2 * EMBED])
        for p in range(P):
            o_ref[p, :, 0:BH] = zlo[:, p * BH:(p + 1) * BH]
            o_ref[p, :, BH:B] = zhi[:, p * BH:(p + 1) * BH]

    in_specs = [
        pl.BlockSpec((P * BH, 2 * EMBED), lambda l: (l, 0)),
        pl.BlockSpec((EMBED, EMBED), lambda l: (0, 0)),
        pl.BlockSpec((EMBED, 1), lambda l: (0, 0)),
    ]
    args = [g_part, W2, b2.reshape(EMBED, 1)]
    io_alias = {}
    if prev is not None:
        in_specs.append(pl.BlockSpec(memory_space=pl.ANY))
        args.append(prev)
        io_alias = {3: 0}
    return pl.pallas_call(
        body,
        grid=(Lp // P,),
        in_specs=in_specs,
        out_specs=pl.BlockSpec((P, EMBED, B),
                               lambda l, s=s: (l + s * (Lp // P), 0, 0)),
        out_shape=jax.ShapeDtypeStruct((L, EMBED, B), jnp.float32),
        input_output_aliases=io_alias,
    )(*args)


def kernel(idxs, table, W1, b1, W2, b2):
    B, L = idxs.shape
    t1w = _table_w1(table, W1, b1)
    t1 = t1w.reshape(VOCAB, EMBED)
    v = idxs.astype(jnp.int32)
    r = 2 * jnp.where(v < HALF, v, v - HALF) + (v >= HALF).astype(jnp.int32)
    rT = r.T
    idx_lo = rT[:, 0:B // 2].reshape(-1)
    idx_hi = rT[:, B // 2:B].reshape(-1)
    NSPLIT = 2
    Lp = L // NSPLIT
    lines_p = Lp * (B // 2)
    out_phys = None
    for s in range(NSPLIT):
        sl = slice(s * lines_p, (s + 1) * lines_p)
        g_s = _sc_gather_wide(t1, idx_lo[sl], idx_hi[sl])
        out_phys = _final_tc_part(g_s, W2, b2, B, L, Lp, s, out_phys)
    return jnp.transpose(out_phys, (2, 0, 1))


# speedup vs baseline: 1.2303x; 1.0444x over previous
import functools

import jax
import jax.numpy as jnp
from jax import lax
from jax.experimental import pallas as pl
from jax.experimental.pallas import tpu as pltpu
from jax.experimental.pallas import tpu_sc as plsc

VOCAB = 100000
FRONT = 256
EMBED = 64
HALF = VOCAB // 2

_NC = 2
_NS = 16
_NW = _NC * _NS


def _table_w1(table, W1, b1):
    BM = 5000
    grid = (HALF // BM,)

    def body(xlo_ref, xhi_ref, w1_ref, b1_ref, o_ref):
        def f(x):
            return lax.dot_general(x, w1_ref[:], (((1,), (1,)), ((), ())),
                                   preferred_element_type=jnp.float32) + b1_ref[:]
        o_ref[:, 0:EMBED] = f(xlo_ref[:])
        o_ref[:, EMBED:2 * EMBED] = f(xhi_ref[:])

    return pl.pallas_call(
        body,
        grid=grid,
        in_specs=[
            pl.BlockSpec((BM, FRONT), lambda i: (i, 0)),
            pl.BlockSpec((BM, FRONT), lambda i: (i + HALF // BM, 0)),
            pl.BlockSpec((EMBED, FRONT), lambda i: (0, 0)),
            pl.BlockSpec((1, EMBED), lambda i: (0, 0)),
        ],
        out_specs=pl.BlockSpec((BM, 2 * EMBED), lambda i: (i, 0)),
        out_shape=jax.ShapeDtypeStruct((HALF, 2 * EMBED), jnp.float32),
    )(table, table, W1, b1.reshape(1, EMBED))


def _sc_gather_wide(t1, idx_lo, idx_hi):
    lines = idx_lo.shape[0]
    l_per_w = lines // _NW
    C = min(1600, l_per_w)
    n_sub = l_per_w // C

    mesh = plsc.VectorSubcoreMesh(core_axis_name="c", subcore_axis_name="s")

    @functools.partial(
        pl.kernel,
        mesh=mesh,
        out_type=jax.ShapeDtypeStruct((lines, 2 * EMBED), jnp.float32),
        scratch_types=[
            pltpu.VMEM((C,), jnp.int32),
            pltpu.VMEM((C, EMBED), jnp.float32),
            pltpu.SemaphoreType.DMA,
        ],
        compiler_params=pltpu.CompilerParams(use_tc_tiling_on_sc=False),
    )
    def k(t1_hbm, ilo_hbm, ihi_hbm, g_hbm, idx_v, rows_v, sem):
        wid = lax.axis_index("s") * _NC + lax.axis_index("c")
        lbase = wid * l_per_w
        for half, src in ((0, ilo_hbm), (1, ihi_hbm)):
            for c in range(n_sub):
                line0 = lbase + c * C
                pltpu.sync_copy(src.at[pl.ds(line0, C)], idx_v)
                pltpu.async_copy(t1_hbm.at[idx_v], rows_v, sem).wait()
                pltpu.sync_copy(
                    rows_v,
                    g_hbm.at[pl.ds(line0, C), pl.ds(half * EMBED, EMBED)])

    return k(t1, idx_lo, idx_hi)


def _final_tc_part(g_part, W2, b2, B, L, Lp, s, prev):
    BH = B // 2
    P = 5

    def body(x_ref, w2_ref, b2_ref, *rest):
        o_ref = rest[-1]
        x = x_ref[:]
        g = x * 0.5 * (1.0 + lax.erf(x * (2.0 ** -0.5)))
        def f(gh):
            return lax.dot_general(w2_ref[:], gh, (((1,), (1,)), ((), ())),
                                   preferred_element_type=jnp.float32) + b2_ref[:]
        zlo = f(g[:, 0:EMBED])
        zhi = f(g[:, EMBED:2 * EMBED])
        for p in range(P):
            o_ref[p, :, 0:BH] = zlo[:, p * BH:(p + 1) * BH]
            o_ref[p, :, BH:B] = zhi[:, p * BH:(p + 1) * BH]

    in_specs = [
        pl.BlockSpec((P * BH, 2 * EMBED), lambda l: (l, 0)),
        pl.BlockSpec((EMBED, EMBED), lambda l: (0, 0)),
        pl.BlockSpec((EMBED, 1), lambda l: (0, 0)),
    ]
    args = [g_part, W2, b2.reshape(EMBED, 1)]
    io_alias = {}
    if prev is not None:
        in_specs.append(pl.BlockSpec(memory_space=pl.ANY))
        args.append(prev)
        io_alias = {3: 0}
    return pl.pallas_call(
        body,
        grid=(Lp // P,),
        in_specs=in_specs,
        out_specs=pl.BlockSpec((P, EMBED, B),
                               lambda l, s=s: (l + s * (Lp // P), 0, 0)),
        out_shape=jax.ShapeDtypeStruct((L, EMBED, B), jnp.float32),
        input_output_aliases=io_alias,
    )(*args)


def kernel(idxs, table, W1, b1, W2, b2):
    B, L = idxs.shape
    t1w = _table_w1(table, W1, b1)
    t1 = t1w.reshape(VOCAB, EMBED)
    v = idxs.astype(jnp.int32)
    r = 2 * jnp.where(v < HALF, v, v - HALF) + (v >= HALF).astype(jnp.int32)
    rT = r.T
    idx_lo = rT[:, 0:B // 2].reshape(-1)
    idx_hi = rT[:, B // 2:B].reshape(-1)
    NSPLIT = 2
    Lp = L // NSPLIT
    lines_p = Lp * (B // 2)
    out_phys = None
    for s in range(NSPLIT):
        sl = slice(s * lines_p, (s + 1) * lines_p)
        g_s = _sc_gather_wide(t1, idx_lo[sl], idx_hi[sl])
        out_phys = _final_tc_part(g_s, W2, b2, B, L, Lp, s, out_phys)
    return jnp.transpose(out_phys, (2, 0, 1))
